# baseline (speedup 1.0000x reference)
import functools

import jax
import jax.numpy as jnp
from jax import lax
from jax.experimental import pallas as pl
from jax.experimental.pallas import tpu as pltpu
from jax.experimental.pallas import tpu_sc as plsc

TLEN, B, SLEN, D, V, CV = 64, 16, 400, 512, 50000, 400
PAD, UNK, IGNORE, EPS = 1, 0, -100, 1e-20
N = TLEN * B

VC = 2048
NCHUNK = (V + VC - 1) // VC
NEG = -1e30

NW = 32
RPW = N // NW
L = 16

_GDN = lax.GatherDimensionNumbers(offset_dims=(), collapsed_slice_dims=(0,),
                                  start_index_map=(0,))


def _lane_sum(acc, lane):
    for sh in (8, 4, 2, 1):
        idx = lax.rem(lane + sh, L)
        acc = acc + lax.gather(acc, idx[:, None], _GDN, (1,),
                               mode=lax.GatherScatterMode.PROMISE_IN_BOUNDS)
    return acc


@functools.lru_cache(maxsize=1)
def _make_sc_copy_base():
    mesh = plsc.VectorSubcoreMesh(core_axis_name="c", subcore_axis_name="s")

    @functools.partial(
        pl.kernel, mesh=mesh,
        compiler_params=pltpu.CompilerParams(use_tc_tiling_on_sc=False),
        out_type=jax.ShapeDtypeStruct((N,), jnp.float32),
        scratch_types=[
            pltpu.VMEM((RPW,), jnp.int32),
            pltpu.VMEM((L, SLEN), jnp.float32),
            pltpu.VMEM((RPW, SLEN), jnp.float32),
            pltpu.VMEM((RPW,), jnp.float32),
        ],
    )
    def sc_copy_base(idsT_hbm, attn_hbm, align_hbm, out_hbm,
                     al_v, ids_v, attn_v, res_v):
        wid = lax.axis_index("s") * 2 + lax.axis_index("c")
        base = wid * RPW
        pltpu.sync_copy(align_hbm.at[pl.ds(base, RPW)], al_v)
        pltpu.sync_copy(attn_hbm.at[pl.ds(base, RPW), :], attn_v)
        pltpu.sync_copy(idsT_hbm, ids_v)
        lane = lax.iota(jnp.int32, L)
        for g in range(RPW // L):
            alg = al_v[pl.ds(g * L, L)].astype(jnp.float32)
            res = jnp.zeros((L,), jnp.float32)
            for r in range(L):
                al_n = alg[r]
                acc = jnp.zeros((L,), jnp.float32)
                for j in range(SLEN // L):
                    sl = pl.ds(j * L, L)
                    acc = acc + jnp.where(ids_v[r, sl] == al_n,
                                          attn_v[g * L + r, sl], 0.0)
                tot = _lane_sum(acc, lane)
                res = jnp.where(lane == r, tot, res)
            res_v[pl.ds(g * L, L)] = res
        pltpu.sync_copy(res_v, out_hbm.at[pl.ds(base, RPW)])

    return sc_copy_base


def _stats_kernel(h_ref, wg_ref, bg_ref, wc_ref, bc_ref, tgt_ref, sm_ref,
                  s_ref, tl_ref, pc_ref, idsT_ref, hbf_ref):
    k = pl.program_id(0)

    @pl.when(k == 0)
    def _init():
        s_ref[...] = jnp.zeros((N, 1), dtype=jnp.float32)
        tl_ref[...] = jnp.zeros((N, 1), dtype=jnp.float32)
        h = h_ref[...]
        hbf_ref[...] = h.astype(jnp.bfloat16)
        z = jnp.sum(h * wc_ref[...], axis=1, keepdims=True) + bc_ref[0, 0]
        pc_ref[...] = jax.nn.sigmoid(z)
        sm = sm_ref[...]
        cidx = jax.lax.broadcasted_iota(jnp.int32, (SLEN, B, CV), 2).astype(
            jnp.float32)
        ids = jnp.sum(sm * cidx, axis=2)
        idsT_ref[...] = lax.transpose(ids, (1, 0))

    whi = wg_ref[...].astype(jnp.bfloat16)
    logits = (jnp.dot(hbf_ref[...], whi, preferred_element_type=jnp.float32)
              + bg_ref[...])
    col = k * VC + jax.lax.broadcasted_iota(jnp.int32, (1, VC), 1)
    valid = (col < V) & (col != PAD)
    logits = jnp.where(valid, logits, NEG)
    s_ref[...] += jnp.sum(jnp.exp(logits), axis=1, keepdims=True)
    tmask = col == tgt_ref[...]
    tl_ref[...] += jnp.sum(jnp.where(tmask, logits, 0.0), axis=1,
                           keepdims=True)


def _combine_kernel(s_ref, tl_ref, pc_ref, cb_ref, tgt_ref, al_ref, out_ref):
    pc = pc_ref[...]
    tg = tgt_ref[...]
    al = al_ref[...]
    vocab_probs = jnp.exp(tl_ref[...]) / s_ref[...] * (1.0 - pc)
    copy_tok = jnp.where(al == UNK, 0.0, cb_ref[...] * pc) + EPS
    non_copy = (al == UNK) | (tg != UNK)
    probs = jnp.where(non_copy, copy_tok + vocab_probs, copy_tok)
    loss = -jnp.log(probs)
    loss = jnp.where(tg == IGNORE, 0.0, loss)
    out_ref[...] = jnp.sum(loss, keepdims=True)


@jax.jit
def kernel(output, copy_attn, src_map, Wg, bg, Wc, bc, target, align):
    hidden = output.reshape(N, D)
    attn = copy_attn.reshape(N, SLEN)
    wcb = Wc.reshape(1, D)
    bc2 = bc.reshape(1, 1)
    bg2 = bg.reshape(1, V)
    tgt = target.reshape(N, 1).astype(jnp.int32)
    al = align.reshape(N, 1).astype(jnp.int32)
    al_flat = align.reshape(N).astype(jnp.int32)

    const2 = lambda shape: pl.BlockSpec(shape, lambda k: (0, 0))
    s, tl, pc, idsT = pl.pallas_call(
        _stats_kernel,
        grid=(NCHUNK,),
        in_specs=[
            const2((N, D)),
            pl.BlockSpec((D, VC), lambda k: (0, k)),
            pl.BlockSpec((1, VC), lambda k: (0, k)),
            const2((1, D)),
            const2((1, 1)),
            const2((N, 1)),
            pl.BlockSpec((SLEN, B, CV), lambda k: (0, 0, 0)),
        ],
        out_specs=[const2((N, 1)), const2((N, 1)), const2((N, 1)),
                   const2((B, SLEN))],
        out_shape=[jax.ShapeDtypeStruct((N, 1), jnp.float32)] * 3
        + [jax.ShapeDtypeStruct((B, SLEN), jnp.float32)],
        scratch_shapes=[
            pltpu.VMEM((N, D), jnp.bfloat16),
        ],
    )(hidden, Wg, bg2, wcb, bc2, tgt, src_map)

    cb = _make_sc_copy_base()(idsT, attn, al_flat)

    out = pl.pallas_call(
        _combine_kernel,
        out_shape=jax.ShapeDtypeStruct((1, 1), jnp.float32),
    )(s, tl, pc, cb.reshape(N, 1), tgt, al)
    return out[0, 0]


# speedup vs baseline: 1.0334x; 1.0334x over previous
import functools

import jax
import jax.numpy as jnp
from jax import lax
from jax.experimental import pallas as pl
from jax.experimental.pallas import tpu as pltpu
from jax.experimental.pallas import tpu_sc as plsc

TLEN, B, SLEN, D, V, CV = 64, 16, 400, 512, 50000, 400
PAD, UNK, IGNORE, EPS = 1, 0, -100, 1e-20
N = TLEN * B

VC = 2048
NCHUNK = (V + VC - 1) // VC
NEG = -1e30

NW = 32
RPW = N // NW
L = 16

_GDN = lax.GatherDimensionNumbers(offset_dims=(), collapsed_slice_dims=(0,),
                                  start_index_map=(0,))


def _lane_sum(acc, lane):
    for sh in (8, 4, 2, 1):
        idx = lax.rem(lane + sh, L)
        acc = acc + lax.gather(acc, idx[:, None], _GDN, (1,),
                               mode=lax.GatherScatterMode.PROMISE_IN_BOUNDS)
    return acc


@functools.lru_cache(maxsize=1)
def _make_sc_copy_base():
    mesh = plsc.VectorSubcoreMesh(core_axis_name="c", subcore_axis_name="s")

    @functools.partial(
        pl.kernel, mesh=mesh,
        compiler_params=pltpu.CompilerParams(use_tc_tiling_on_sc=False),
        out_type=jax.ShapeDtypeStruct((N,), jnp.float32),
        scratch_types=[
            pltpu.VMEM((RPW,), jnp.int32),
            pltpu.VMEM((L, SLEN), jnp.float32),
            pltpu.VMEM((RPW, SLEN), jnp.float32),
            pltpu.VMEM((RPW,), jnp.float32),
        ],
    )
    def sc_copy_base(idsT_hbm, attn_hbm, align_hbm, out_hbm,
                     al_v, ids_v, attn_v, res_v):
        wid = lax.axis_index("s") * 2 + lax.axis_index("c")
        base = wid * RPW
        pltpu.sync_copy(align_hbm.at[pl.ds(base, RPW)], al_v)
        pltpu.sync_copy(attn_hbm.at[pl.ds(base, RPW), :], attn_v)
        pltpu.sync_copy(idsT_hbm, ids_v)
        lane = lax.iota(jnp.int32, L)
        for g in range(RPW // L):
            alg = al_v[pl.ds(g * L, L)].astype(jnp.float32)
            res = jnp.zeros((L,), jnp.float32)
            for r in range(L):
                al_n = alg[r]
                acc = jnp.zeros((L,), jnp.float32)
                for j in range(SLEN // L):
                    sl = pl.ds(j * L, L)
                    acc = acc + jnp.where(ids_v[r, sl] == al_n,
                                          attn_v[g * L + r, sl], 0.0)
                tot = _lane_sum(acc, lane)
                res = jnp.where(lane == r, tot, res)
            res_v[pl.ds(g * L, L)] = res
        pltpu.sync_copy(res_v, out_hbm.at[pl.ds(base, RPW)])

    return sc_copy_base


def _ids_kernel(sm_ref, idsT_ref):
    sm = sm_ref[...]
    cidx = jax.lax.broadcasted_iota(jnp.int32, (SLEN, B, CV), 2).astype(
        jnp.float32)
    ids = jnp.sum(sm * cidx, axis=2)
    idsT_ref[...] = lax.transpose(ids, (1, 0))


def _stats_kernel(h_ref, wg_ref, bg_ref, wc_ref, bc_ref, tgt_ref,
                  s_ref, tl_ref, pc_ref, hbf_ref):
    k = pl.program_id(0)

    @pl.when(k == 0)
    def _init():
        s_ref[...] = jnp.zeros((N, 1), dtype=jnp.float32)
        tl_ref[...] = jnp.zeros((N, 1), dtype=jnp.float32)
        h = h_ref[...]
        hbf_ref[...] = h.astype(jnp.bfloat16)
        z = jnp.sum(h * wc_ref[...], axis=1, keepdims=True) + bc_ref[0, 0]
        pc_ref[...] = jax.nn.sigmoid(z)

    whi = wg_ref[...].astype(jnp.bfloat16)
    logits = (jnp.dot(hbf_ref[...], whi, preferred_element_type=jnp.float32)
              + bg_ref[...])
    col = k * VC + jax.lax.broadcasted_iota(jnp.int32, (1, VC), 1)
    valid = (col < V) & (col != PAD)
    logits = jnp.where(valid, logits, NEG)
    s_ref[...] += jnp.sum(jnp.exp(logits), axis=1, keepdims=True)
    tmask = col == tgt_ref[...]
    tl_ref[...] += jnp.sum(jnp.where(tmask, logits, 0.0), axis=1,
                           keepdims=True)


def _combine_kernel(s_ref, tl_ref, pc_ref, cb_ref, tgt_ref, al_ref, out_ref):
    pc = pc_ref[...]
    tg = tgt_ref[...]
    al = al_ref[...]
    vocab_probs = jnp.exp(tl_ref[...]) / s_ref[...] * (1.0 - pc)
    copy_tok = jnp.where(al == UNK, 0.0, cb_ref[...] * pc) + EPS
    non_copy = (al == UNK) | (tg != UNK)
    probs = jnp.where(non_copy, copy_tok + vocab_probs, copy_tok)
    loss = -jnp.log(probs)
    loss = jnp.where(tg == IGNORE, 0.0, loss)
    out_ref[...] = jnp.sum(loss, keepdims=True)


@jax.jit
def kernel(output, copy_attn, src_map, Wg, bg, Wc, bc, target, align):
    hidden = output.reshape(N, D)
    attn = copy_attn.reshape(N, SLEN)
    wcb = Wc.reshape(1, D)
    bc2 = bc.reshape(1, 1)
    bg2 = bg.reshape(1, V)
    tgt = target.reshape(N, 1).astype(jnp.int32)
    al = align.reshape(N, 1).astype(jnp.int32)
    al_flat = align.reshape(N).astype(jnp.int32)

    idsT = pl.pallas_call(
        _ids_kernel,
        out_shape=jax.ShapeDtypeStruct((B, SLEN), jnp.float32),
    )(src_map)
    cb = _make_sc_copy_base()(idsT, attn, al_flat)

    const2 = lambda shape: pl.BlockSpec(shape, lambda k: (0, 0))
    s, tl, pc = pl.pallas_call(
        _stats_kernel,
        grid=(NCHUNK,),
        in_specs=[
            const2((N, D)),
            pl.BlockSpec((D, VC), lambda k: (0, k)),
            pl.BlockSpec((1, VC), lambda k: (0, k)),
            const2((1, D)),
            const2((1, 1)),
            const2((N, 1)),
        ],
        out_specs=[const2((N, 1)), const2((N, 1)), const2((N, 1))],
        out_shape=[jax.ShapeDtypeStruct((N, 1), jnp.float32)] * 3,
        scratch_shapes=[
            pltpu.VMEM((N, D), jnp.bfloat16),
        ],
    )(hidden, Wg, bg2, wcb, bc2, tgt)

    out = pl.pallas_call(
        _combine_kernel,
        out_shape=jax.ShapeDtypeStruct((1, 1), jnp.float32),
    )(s, tl, pc, cb.reshape(N, 1), tgt, al)
    return out[0, 0]
